# Initial kernel scaffold; baseline (speedup 1.0000x reference)
import jax
import jax.numpy as jnp
from jax.experimental import pallas as pl


def kernel(xyz, features):
    raise NotImplementedError("write your pallas kernel here")


# speedup vs baseline: 1.0180x; 1.0180x over previous
import functools

import jax
import jax.numpy as jnp
from jax import lax
from jax.experimental import pallas as pl
from jax.experimental.pallas import tpu as pltpu
from jax.experimental.pallas import tpu_sc as plsc

_NUM_GROUPS = 512
_GROUP_SIZE = 32

_NC, _NS = 2, 16
_NW = _NC * _NS


def _gather_rows_sc(table, idx, chunk=1024):
    M = idx.shape[0]
    D = table.shape[1]
    b_per_w = M // _NW
    nchunks = b_per_w // chunk
    assert b_per_w % chunk == 0 and M % _NW == 0
    mesh = plsc.VectorSubcoreMesh(core_axis_name="c", subcore_axis_name="s")

    @functools.partial(
        pl.kernel,
        out_type=jax.ShapeDtypeStruct((M, D), jnp.float32),
        mesh=mesh,
        scratch_types=[
            pltpu.VMEM((b_per_w,), jnp.int32),
            pltpu.VMEM((chunk, D), jnp.float32),
            pltpu.SemaphoreType.DMA,
        ],
        compiler_params=pltpu.CompilerParams(use_tc_tiling_on_sc=False),
    )
    def k(table_hbm, idx_hbm, out_hbm, idx_v, rows_v, sem):
        wid = lax.axis_index("s") * _NC + lax.axis_index("c")
        base = wid * b_per_w
        pltpu.sync_copy(idx_hbm.at[pl.ds(base, b_per_w)], idx_v)
        for c in range(nchunks):
            pltpu.async_copy(
                table_hbm.at[idx_v.at[pl.ds(c * chunk, chunk)]], rows_v, sem
            ).wait()
            pltpu.sync_copy(rows_v, out_hbm.at[pl.ds(base + c * chunk, chunk)])

    return k(table, idx)


def _fps(xyz, K):
    B, N, _ = xyz.shape

    def body(i, state):
        idx, dists, farthest = state
        idx = idx.at[:, i].set(farthest)
        centroid = jnp.take_along_axis(xyz, farthest[:, None, None], axis=1)
        d = jnp.sum((xyz - centroid) ** 2, axis=-1)
        dists = jnp.minimum(dists, d)
        farthest = jnp.argmax(dists, axis=-1).astype(jnp.int32)
        return idx, dists, farthest

    idx0 = jnp.zeros((B, K), dtype=jnp.int32)
    dists0 = jnp.full((B, N), jnp.inf, dtype=jnp.float32)
    far0 = jnp.zeros((B,), dtype=jnp.int32)
    idx, _, _ = jax.lax.fori_loop(0, K, body, (idx0, dists0, far0))
    return idx


def _knn(centers, key_pts, k):
    q2 = jnp.sum(centers ** 2, axis=-1)[:, :, None]
    k2 = jnp.sum(key_pts ** 2, axis=-1)[:, None, :]
    d2 = q2 + k2 - 2.0 * jnp.einsum('bgd,bnd->bgn', centers, key_pts)
    d = jnp.sqrt(jnp.maximum(d2, 0.0))
    neg_d, idx = jax.lax.top_k(-d, k)
    return -neg_d, idx


def kernel(xyz, features):
    B, N, _ = xyz.shape
    C = features.shape[-1]
    G, K = _NUM_GROUPS, _GROUP_SIZE

    fps_idx = _fps(xyz, G)
    centers = jnp.take_along_axis(xyz, fps_idx[:, :, None], axis=1)
    _, knn_idx = _knn(centers, xyz, K)

    batch_offset = (jnp.arange(B, dtype=knn_idx.dtype) * N).reshape(-1, 1, 1)
    knn_idx_flat = (knn_idx + batch_offset).reshape(-1)

    nbr_feats = _gather_rows_sc(features.reshape(-1, C), knn_idx_flat)
    nbr_feats = nbr_feats.reshape(B, G, K, C)
    nbr_xyz = xyz.reshape(-1, 3)[knn_idx_flat].reshape(B, G, K, 3)
    nbr_xyz = nbr_xyz - centers[:, :, None, :]
    group_feats = jnp.concatenate([nbr_xyz, nbr_feats], axis=-1)
    return group_feats, centers, knn_idx


# speedup vs baseline: 1.9290x; 1.8948x over previous
import functools

import jax
import jax.numpy as jnp
from jax import lax
from jax.experimental import pallas as pl
from jax.experimental.pallas import tpu as pltpu
from jax.experimental.pallas import tpu_sc as plsc

_NUM_GROUPS = 512
_GROUP_SIZE = 32

_NC, _NS = 2, 16
_NW = _NC * _NS


def _gather_rows_sc(table, idx, chunk=1024):
    M = idx.shape[0]
    D = table.shape[1]
    b_per_w = M // _NW
    nchunks = b_per_w // chunk
    assert b_per_w % chunk == 0 and M % _NW == 0
    mesh = plsc.VectorSubcoreMesh(core_axis_name="c", subcore_axis_name="s")

    @functools.partial(
        pl.kernel,
        out_type=jax.ShapeDtypeStruct((M, D), jnp.float32),
        mesh=mesh,
        scratch_types=[
            pltpu.VMEM((b_per_w,), jnp.int32),
            pltpu.VMEM((chunk, D), jnp.float32),
            pltpu.SemaphoreType.DMA,
        ],
        compiler_params=pltpu.CompilerParams(use_tc_tiling_on_sc=False),
    )
    def k(table_hbm, idx_hbm, out_hbm, idx_v, rows_v, sem):
        wid = lax.axis_index("s") * _NC + lax.axis_index("c")
        base = wid * b_per_w
        pltpu.sync_copy(idx_hbm.at[pl.ds(base, b_per_w)], idx_v)
        for c in range(nchunks):
            pltpu.async_copy(
                table_hbm.at[idx_v.at[pl.ds(c * chunk, chunk)]], rows_v, sem
            ).wait()
            pltpu.sync_copy(rows_v, out_hbm.at[pl.ds(base + c * chunk, chunk)])

    return k(table, idx)


def _fps_centers_tc(xs, ys, zs, G, interpret=False):
    B, N = xs.shape

    def body(xs_ref, ys_ref, zs_ref, cx_ref, cy_ref, cz_ref):
        xs_, ys_, zs_ = xs_ref[...], ys_ref[...], zs_ref[...]
        iota_n = lax.broadcasted_iota(jnp.int32, (B, N), 1)
        iota_g = lax.broadcasted_iota(jnp.int32, (B, G), 1)

        def step(i, state):
            dists, far, cxa, cya, cza = state
            mask = iota_n == far
            cx = jnp.sum(jnp.where(mask, xs_, 0.0), axis=1, keepdims=True)
            cy = jnp.sum(jnp.where(mask, ys_, 0.0), axis=1, keepdims=True)
            cz = jnp.sum(jnp.where(mask, zs_, 0.0), axis=1, keepdims=True)
            gm = iota_g == i
            cxa = cxa + jnp.where(gm, cx, 0.0)
            cya = cya + jnp.where(gm, cy, 0.0)
            cza = cza + jnp.where(gm, cz, 0.0)
            dx = xs_ - cx
            dy = ys_ - cy
            dz = zs_ - cz
            d = dx * dx + dy * dy + dz * dz
            dists = jnp.minimum(dists, d)
            m = jnp.max(dists, axis=1, keepdims=True)
            far = jnp.min(jnp.where(dists == m, iota_n, N), axis=1,
                          keepdims=True)
            return dists, far, cxa, cya, cza

        dists0 = jnp.full((B, N), jnp.inf, jnp.float32)
        far0 = jnp.zeros((B, 1), jnp.int32)
        z = jnp.zeros((B, G), jnp.float32)
        _, _, cxa, cya, cza = lax.fori_loop(0, G, step,
                                            (dists0, far0, z, z, z))
        cx_ref[...] = cxa
        cy_ref[...] = cya
        cz_ref[...] = cza

    return pl.pallas_call(
        body,
        out_shape=[jax.ShapeDtypeStruct((B, G), jnp.float32)] * 3,
        interpret=interpret,
    )(xs, ys, zs)


def _knn(centers, key_pts, k):
    q2 = jnp.sum(centers ** 2, axis=-1)[:, :, None]
    k2 = jnp.sum(key_pts ** 2, axis=-1)[:, None, :]
    d2 = q2 + k2 - 2.0 * jnp.einsum('bgd,bnd->bgn', centers, key_pts)
    d = jnp.sqrt(jnp.maximum(d2, 0.0))
    neg_d, idx = jax.lax.top_k(-d, k)
    return -neg_d, idx


def kernel(xyz, features):
    B, N, _ = xyz.shape
    C = features.shape[-1]
    G, K = _NUM_GROUPS, _GROUP_SIZE

    xt = xyz.transpose(0, 2, 1)
    cx, cy, cz = _fps_centers_tc(xt[:, 0], xt[:, 1], xt[:, 2], G)
    centers = jnp.stack([cx, cy, cz], axis=-1)
    _, knn_idx = _knn(centers, xyz, K)

    batch_offset = (jnp.arange(B, dtype=knn_idx.dtype) * N).reshape(-1, 1, 1)
    knn_idx_flat = (knn_idx + batch_offset).reshape(-1)

    nbr_feats = _gather_rows_sc(features.reshape(-1, C), knn_idx_flat)
    nbr_feats = nbr_feats.reshape(B, G, K, C)
    nbr_xyz = xyz.reshape(-1, 3)[knn_idx_flat].reshape(B, G, K, 3)
    nbr_xyz = nbr_xyz - centers[:, :, None, :]
    group_feats = jnp.concatenate([nbr_xyz, nbr_feats], axis=-1)
    return group_feats, centers, knn_idx


# speedup vs baseline: 5.6126x; 2.9095x over previous
import functools

import jax
import jax.numpy as jnp
from jax import lax
from jax.experimental import pallas as pl
from jax.experimental.pallas import tpu as pltpu
from jax.experimental.pallas import tpu_sc as plsc

_NUM_GROUPS = 512
_GROUP_SIZE = 32

_NC, _NS = 2, 16
_NW = _NC * _NS


def _gather_rows_sc(feats, xyzp, idx, chunk=512):
    M = idx.shape[0]
    Df, Dx = feats.shape[1], xyzp.shape[1]
    b_per_w = M // _NW
    nchunks = b_per_w // chunk
    assert b_per_w % chunk == 0 and M % _NW == 0
    mesh = plsc.VectorSubcoreMesh(core_axis_name="c", subcore_axis_name="s")

    @functools.partial(
        pl.kernel,
        out_type=[
            jax.ShapeDtypeStruct((M, Df), jnp.float32),
            jax.ShapeDtypeStruct((M, Dx), jnp.float32),
        ],
        mesh=mesh,
        scratch_types=[
            pltpu.VMEM((b_per_w,), jnp.int32),
            pltpu.VMEM((2, chunk, Df), jnp.float32),
            pltpu.VMEM((2, chunk, Dx), jnp.float32),
            pltpu.SemaphoreType.DMA,
            pltpu.SemaphoreType.DMA,
        ],
        compiler_params=pltpu.CompilerParams(use_tc_tiling_on_sc=False),
    )
    def k(feats_hbm, xyzp_hbm, idx_hbm, of_hbm, ox_hbm,
          idx_v, frows, xrows, semf, semx):
        wid = lax.axis_index("s") * _NC + lax.axis_index("c")
        base = wid * b_per_w
        pltpu.sync_copy(idx_hbm.at[pl.ds(base, b_per_w)], idx_v)
        copies = [None] * nchunks
        for c in range(nchunks):
            ix = idx_v.at[pl.ds(c * chunk, chunk)]
            buf = c % 2
            copies[c] = (
                pltpu.async_copy(feats_hbm.at[ix], frows.at[buf], semf),
                pltpu.async_copy(xyzp_hbm.at[ix], xrows.at[buf], semx),
            )
            if c > 0:
                prev = c - 1
                for cp in copies[prev]:
                    cp.wait()
                sl = pl.ds(base + prev * chunk, chunk)
                pltpu.sync_copy(frows.at[prev % 2], of_hbm.at[sl])
                pltpu.sync_copy(xrows.at[prev % 2], ox_hbm.at[sl])
        last = nchunks - 1
        for cp in copies[last]:
            cp.wait()
        sl = pl.ds(base + last * chunk, chunk)
        pltpu.sync_copy(frows.at[last % 2], of_hbm.at[sl])
        pltpu.sync_copy(xrows.at[last % 2], ox_hbm.at[sl])

    return k(feats, xyzp, idx)


def _fps_centers_tc(xs, ys, zs, G, interpret=False):
    B, N = xs.shape

    def body(xs_ref, ys_ref, zs_ref, cx_ref, cy_ref, cz_ref):
        xs_, ys_, zs_ = xs_ref[...], ys_ref[...], zs_ref[...]
        iota_n = lax.broadcasted_iota(jnp.int32, (B, N), 1)
        iota_g = lax.broadcasted_iota(jnp.int32, (B, G), 1)

        def step(i, state):
            dists, far, cxa, cya, cza = state
            mask = iota_n == far
            cx = jnp.sum(jnp.where(mask, xs_, 0.0), axis=1, keepdims=True)
            cy = jnp.sum(jnp.where(mask, ys_, 0.0), axis=1, keepdims=True)
            cz = jnp.sum(jnp.where(mask, zs_, 0.0), axis=1, keepdims=True)
            gm = iota_g == i
            cxa = cxa + jnp.where(gm, cx, 0.0)
            cya = cya + jnp.where(gm, cy, 0.0)
            cza = cza + jnp.where(gm, cz, 0.0)
            dx = xs_ - cx
            dy = ys_ - cy
            dz = zs_ - cz
            d = dx * dx + dy * dy + dz * dz
            dists = jnp.minimum(dists, d)
            m = jnp.max(dists, axis=1, keepdims=True)
            far = jnp.min(jnp.where(dists == m, iota_n, N), axis=1,
                          keepdims=True)
            return dists, far, cxa, cya, cza

        dists0 = jnp.full((B, N), jnp.inf, jnp.float32)
        far0 = jnp.zeros((B, 1), jnp.int32)
        z = jnp.zeros((B, G), jnp.float32)
        _, _, cxa, cya, cza = lax.fori_loop(0, G, step,
                                            (dists0, far0, z, z, z))
        cx_ref[...] = cxa
        cy_ref[...] = cya
        cz_ref[...] = cza

    return pl.pallas_call(
        body,
        out_shape=[jax.ShapeDtypeStruct((B, G), jnp.float32)] * 3,
        interpret=interpret,
    )(xs, ys, zs)


def _knn_idx_tc(xyz, ct, K, gt=128, interpret=False):
    B, N, _ = xyz.shape
    G = ct.shape[2]

    def body(p_ref, c_ref, out_ref):
        p = p_ref[0]
        c = c_ref[0]
        k2 = jnp.sum(p * p, axis=1, keepdims=True)
        q2 = jnp.sum(c * c, axis=0, keepdims=True)
        dot = jax.lax.dot_general(
            p, c, dimension_numbers=(((1,), (0,)), ((), ())),
            preferred_element_type=jnp.float32)
        d2 = q2 + k2 - 2.0 * dot
        d = jnp.sqrt(jnp.maximum(d2, 0.0))
        iota_n = lax.broadcasted_iota(jnp.int32, (N, gt), 0)
        iota_k = lax.broadcasted_iota(jnp.int32, (K, gt), 0)

        def step(k, state):
            d_cur, acc = state
            m = jnp.min(d_cur, axis=0, keepdims=True)
            hit = d_cur == m
            idx = jnp.min(jnp.where(hit, iota_n, N), axis=0,
                          keepdims=True)
            acc = acc + jnp.where(iota_k == k, idx, 0)
            d_cur = jnp.where(iota_n == idx, jnp.inf, d_cur)
            return d_cur, acc

        acc0 = jnp.zeros((K, gt), jnp.int32)
        _, acc = lax.fori_loop(0, K, step, (d, acc0))
        out_ref[0] = acc

    return pl.pallas_call(
        body,
        grid=(B, G // gt),
        in_specs=[
            pl.BlockSpec((1, N, 3), lambda b, g: (b, 0, 0)),
            pl.BlockSpec((1, 3, gt), lambda b, g: (b, 0, g)),
        ],
        out_specs=pl.BlockSpec((1, K, gt), lambda b, g: (b, 0, g)),
        out_shape=jax.ShapeDtypeStruct((B, K, G), jnp.int32),
        compiler_params=pltpu.CompilerParams(
            vmem_limit_bytes=110 * 1024 * 1024),
        interpret=interpret,
    )(xyz, ct)


def kernel(xyz, features):
    B, N, _ = xyz.shape
    C = features.shape[-1]
    G, K = _NUM_GROUPS, _GROUP_SIZE

    xt = xyz.transpose(0, 2, 1)
    cx, cy, cz = _fps_centers_tc(xt[:, 0], xt[:, 1], xt[:, 2], G)
    centers = jnp.stack([cx, cy, cz], axis=-1)
    ct = jnp.stack([cx, cy, cz], axis=1)
    knn_idx = _knn_idx_tc(xyz, ct, K).transpose(0, 2, 1)

    batch_offset = (jnp.arange(B, dtype=knn_idx.dtype) * N).reshape(-1, 1, 1)
    knn_idx_flat = (knn_idx + batch_offset).reshape(-1)

    xyzp = jnp.pad(xyz.reshape(-1, 3), ((0, 0), (0, 13)))
    nbr_feats, nbr_xyzp = _gather_rows_sc(
        features.reshape(-1, C), xyzp, knn_idx_flat)
    nbr_feats = nbr_feats.reshape(B, G, K, C)
    nbr_xyz = nbr_xyzp[:, :3].reshape(B, G, K, 3) - centers[:, :, None, :]
    group_feats = jnp.concatenate([nbr_xyz, nbr_feats], axis=-1)
    return group_feats, centers, knn_idx


# speedup vs baseline: 6.5660x; 1.1699x over previous
import functools

import jax
import jax.numpy as jnp
from jax import lax
from jax.experimental import pallas as pl
from jax.experimental.pallas import tpu as pltpu
from jax.experimental.pallas import tpu_sc as plsc

_NUM_GROUPS = 512
_GROUP_SIZE = 32

_NC, _NS = 2, 16
_NW = _NC * _NS


def _gather_rows_sc(feats, xyzp, idx, chunk=512):
    M = idx.shape[0]
    Df, Dx = feats.shape[1], xyzp.shape[1]
    b_per_w = M // _NW
    nchunks = b_per_w // chunk
    assert b_per_w % chunk == 0 and M % _NW == 0
    mesh = plsc.VectorSubcoreMesh(core_axis_name="c", subcore_axis_name="s")

    @functools.partial(
        pl.kernel,
        out_type=[
            jax.ShapeDtypeStruct((M, Df), jnp.float32),
            jax.ShapeDtypeStruct((M, Dx), jnp.float32),
        ],
        mesh=mesh,
        scratch_types=[
            pltpu.VMEM((b_per_w,), jnp.int32),
            pltpu.VMEM((2, chunk, Df), jnp.float32),
            pltpu.VMEM((2, chunk, Dx), jnp.float32),
            pltpu.SemaphoreType.DMA,
            pltpu.SemaphoreType.DMA,
        ],
        compiler_params=pltpu.CompilerParams(use_tc_tiling_on_sc=False),
    )
    def k(feats_hbm, xyzp_hbm, idx_hbm, of_hbm, ox_hbm,
          idx_v, frows, xrows, semf, semx):
        wid = lax.axis_index("s") * _NC + lax.axis_index("c")
        base = wid * b_per_w
        pltpu.sync_copy(idx_hbm.at[pl.ds(base, b_per_w)], idx_v)
        copies = [None] * nchunks
        for c in range(nchunks):
            ix = idx_v.at[pl.ds(c * chunk, chunk)]
            buf = c % 2
            copies[c] = (
                pltpu.async_copy(feats_hbm.at[ix], frows.at[buf], semf),
                pltpu.async_copy(xyzp_hbm.at[ix], xrows.at[buf], semx),
            )
            if c > 0:
                prev = c - 1
                for cp in copies[prev]:
                    cp.wait()
                sl = pl.ds(base + prev * chunk, chunk)
                pltpu.sync_copy(frows.at[prev % 2], of_hbm.at[sl])
                pltpu.sync_copy(xrows.at[prev % 2], ox_hbm.at[sl])
        last = nchunks - 1
        for cp in copies[last]:
            cp.wait()
        sl = pl.ds(base + last * chunk, chunk)
        pltpu.sync_copy(frows.at[last % 2], of_hbm.at[sl])
        pltpu.sync_copy(xrows.at[last % 2], ox_hbm.at[sl])

    return k(feats, xyzp, idx)


def _fps_centers_tc(xs, ys, zs, G, interpret=False):
    B, N = xs.shape

    def body(xs_ref, ys_ref, zs_ref, cx_ref, cy_ref, cz_ref):
        xs_, ys_, zs_ = xs_ref[...], ys_ref[...], zs_ref[...]
        iota_n = lax.broadcasted_iota(jnp.int32, (B, N), 1)
        iota_g = lax.broadcasted_iota(jnp.int32, (B, G), 1)

        def step(i, state):
            dists, far, cxa, cya, cza = state
            mask = iota_n == far
            cx = jnp.sum(jnp.where(mask, xs_, 0.0), axis=1, keepdims=True)
            cy = jnp.sum(jnp.where(mask, ys_, 0.0), axis=1, keepdims=True)
            cz = jnp.sum(jnp.where(mask, zs_, 0.0), axis=1, keepdims=True)
            gm = iota_g == i
            cxa = cxa + jnp.where(gm, cx, 0.0)
            cya = cya + jnp.where(gm, cy, 0.0)
            cza = cza + jnp.where(gm, cz, 0.0)
            dx = xs_ - cx
            dy = ys_ - cy
            dz = zs_ - cz
            d = dx * dx + dy * dy + dz * dz
            dists = jnp.minimum(dists, d)
            m = jnp.max(dists, axis=1, keepdims=True)
            far = jnp.min(jnp.where(dists == m, iota_n, N), axis=1,
                          keepdims=True)
            return dists, far, cxa, cya, cza

        dists0 = jnp.full((B, N), jnp.inf, jnp.float32)
        far0 = jnp.zeros((B, 1), jnp.int32)
        z = jnp.zeros((B, G), jnp.float32)
        _, _, cxa, cya, cza = lax.fori_loop(0, G, step,
                                            (dists0, far0, z, z, z))
        cx_ref[...] = cxa
        cy_ref[...] = cya
        cz_ref[...] = cza

    return pl.pallas_call(
        body,
        out_shape=[jax.ShapeDtypeStruct((B, G), jnp.float32)] * 3,
        interpret=interpret,
    )(xs, ys, zs)


def _knn_idx_tc(xyz, ct, K, gt=128, interpret=False):
    B, N, _ = xyz.shape
    G = ct.shape[2]

    def body(p_ref, c_ref, out_ref, d_ref):
        p = p_ref[0]
        c = c_ref[0]
        k2 = jnp.sum(p * p, axis=1, keepdims=True)
        q2 = jnp.sum(c * c, axis=0, keepdims=True)
        dot = jax.lax.dot_general(
            p, c, dimension_numbers=(((1,), (0,)), ((), ())),
            preferred_element_type=jnp.float32)
        d2 = q2 + k2 - 2.0 * dot
        d_ref[...] = jnp.sqrt(jnp.maximum(d2, 0.0))
        iota_n = lax.broadcasted_iota(jnp.int32, (N, gt), 0)
        iota_k = lax.broadcasted_iota(jnp.int32, (K, gt), 0)

        def step(k, acc):
            d_cur = d_ref[...]
            m = jnp.min(d_cur, axis=0, keepdims=True)
            hit = d_cur == m
            idx = jnp.min(jnp.where(hit, iota_n, N), axis=0,
                          keepdims=True)
            acc = acc + jnp.where(iota_k == k, idx, 0)
            d_ref[...] = jnp.where(iota_n == idx, jnp.inf, d_cur)
            return acc

        acc0 = jnp.zeros((K, gt), jnp.int32)
        out_ref[0] = lax.fori_loop(0, K, step, acc0)

    return pl.pallas_call(
        body,
        grid=(B, G // gt),
        in_specs=[
            pl.BlockSpec((1, N, 3), lambda b, g: (b, 0, 0)),
            pl.BlockSpec((1, 3, gt), lambda b, g: (b, 0, g)),
        ],
        out_specs=pl.BlockSpec((1, K, gt), lambda b, g: (b, 0, g)),
        out_shape=jax.ShapeDtypeStruct((B, K, G), jnp.int32),
        scratch_shapes=[pltpu.VMEM((N, gt), jnp.float32)],
        compiler_params=pltpu.CompilerParams(
            vmem_limit_bytes=110 * 1024 * 1024),
        interpret=interpret,
    )(xyz, ct)


def kernel(xyz, features):
    B, N, _ = xyz.shape
    C = features.shape[-1]
    G, K = _NUM_GROUPS, _GROUP_SIZE

    xt = xyz.transpose(0, 2, 1)
    cx, cy, cz = _fps_centers_tc(xt[:, 0], xt[:, 1], xt[:, 2], G)
    centers = jnp.stack([cx, cy, cz], axis=-1)
    ct = jnp.stack([cx, cy, cz], axis=1)
    knn_idx = _knn_idx_tc(xyz, ct, K).transpose(0, 2, 1)

    batch_offset = (jnp.arange(B, dtype=knn_idx.dtype) * N).reshape(-1, 1, 1)
    knn_idx_flat = (knn_idx + batch_offset).reshape(-1)

    xyzp = jnp.pad(xyz.reshape(-1, 3), ((0, 0), (0, 13)))
    nbr_feats, nbr_xyzp = _gather_rows_sc(
        features.reshape(-1, C), xyzp, knn_idx_flat)
    nbr_feats = nbr_feats.reshape(B, G, K, C)
    nbr_xyz = nbr_xyzp[:, :3].reshape(B, G, K, 3) - centers[:, :, None, :]
    group_feats = jnp.concatenate([nbr_xyz, nbr_feats], axis=-1)
    return group_feats, centers, knn_idx


# speedup vs baseline: 8.9135x; 1.3575x over previous
import functools

import jax
import jax.numpy as jnp
from jax import lax
from jax.experimental import pallas as pl
from jax.experimental.pallas import tpu as pltpu
from jax.experimental.pallas import tpu_sc as plsc

_NUM_GROUPS = 512
_GROUP_SIZE = 32

_NC, _NS = 2, 16
_NW = _NC * _NS


def _gather_rows_sc(feats, xyzp, idx, chunk=512):
    M = idx.shape[0]
    Df, Dx = feats.shape[1], xyzp.shape[1]
    b_per_w = M // _NW
    nchunks = b_per_w // chunk
    assert b_per_w % chunk == 0 and M % _NW == 0
    mesh = plsc.VectorSubcoreMesh(core_axis_name="c", subcore_axis_name="s")

    @functools.partial(
        pl.kernel,
        out_type=[
            jax.ShapeDtypeStruct((M, Df), jnp.float32),
            jax.ShapeDtypeStruct((M, Dx), jnp.float32),
        ],
        mesh=mesh,
        scratch_types=[
            pltpu.VMEM((b_per_w,), jnp.int32),
            pltpu.VMEM((2, chunk, Df), jnp.float32),
            pltpu.VMEM((2, chunk, Dx), jnp.float32),
            pltpu.SemaphoreType.DMA,
            pltpu.SemaphoreType.DMA,
        ],
        compiler_params=pltpu.CompilerParams(use_tc_tiling_on_sc=False),
    )
    def k(feats_hbm, xyzp_hbm, idx_hbm, of_hbm, ox_hbm,
          idx_v, frows, xrows, semf, semx):
        wid = lax.axis_index("s") * _NC + lax.axis_index("c")
        base = wid * b_per_w
        pltpu.sync_copy(idx_hbm.at[pl.ds(base, b_per_w)], idx_v)
        copies = [None] * nchunks
        for c in range(nchunks):
            ix = idx_v.at[pl.ds(c * chunk, chunk)]
            buf = c % 2
            copies[c] = (
                pltpu.async_copy(feats_hbm.at[ix], frows.at[buf], semf),
                pltpu.async_copy(xyzp_hbm.at[ix], xrows.at[buf], semx),
            )
            if c > 0:
                prev = c - 1
                for cp in copies[prev]:
                    cp.wait()
                sl = pl.ds(base + prev * chunk, chunk)
                pltpu.sync_copy(frows.at[prev % 2], of_hbm.at[sl])
                pltpu.sync_copy(xrows.at[prev % 2], ox_hbm.at[sl])
        last = nchunks - 1
        for cp in copies[last]:
            cp.wait()
        sl = pl.ds(base + last * chunk, chunk)
        pltpu.sync_copy(frows.at[last % 2], of_hbm.at[sl])
        pltpu.sync_copy(xrows.at[last % 2], ox_hbm.at[sl])

    return k(feats, xyzp, idx)


def _fps_centers_tc(xs, ys, zs, G, interpret=False):
    B, N = xs.shape

    def body(xs_ref, ys_ref, zs_ref, cx_ref, cy_ref, cz_ref):
        xs_, ys_, zs_ = xs_ref[...], ys_ref[...], zs_ref[...]
        iota_n = lax.broadcasted_iota(jnp.int32, (B, N), 1)
        iota_g = lax.broadcasted_iota(jnp.int32, (B, G), 1)

        def step(i, state):
            dists, far, cxa, cya, cza = state
            mask = iota_n == far
            cx = jnp.sum(jnp.where(mask, xs_, 0.0), axis=1, keepdims=True)
            cy = jnp.sum(jnp.where(mask, ys_, 0.0), axis=1, keepdims=True)
            cz = jnp.sum(jnp.where(mask, zs_, 0.0), axis=1, keepdims=True)
            gm = iota_g == i
            cxa = cxa + jnp.where(gm, cx, 0.0)
            cya = cya + jnp.where(gm, cy, 0.0)
            cza = cza + jnp.where(gm, cz, 0.0)
            dx = xs_ - cx
            dy = ys_ - cy
            dz = zs_ - cz
            d = dx * dx + dy * dy + dz * dz
            dists = jnp.minimum(dists, d)
            m = jnp.max(dists, axis=1, keepdims=True)
            far = jnp.min(jnp.where(dists == m, iota_n, N), axis=1,
                          keepdims=True)
            return dists, far, cxa, cya, cza

        dists0 = jnp.full((B, N), jnp.inf, jnp.float32)
        far0 = jnp.zeros((B, 1), jnp.int32)
        z = jnp.zeros((B, G), jnp.float32)
        _, _, cxa, cya, cza = lax.fori_loop(0, G, step,
                                            (dists0, far0, z, z, z))
        cx_ref[...] = cxa
        cy_ref[...] = cya
        cz_ref[...] = cza

    return pl.pallas_call(
        body,
        out_shape=[jax.ShapeDtypeStruct((B, G), jnp.float32)] * 3,
        interpret=interpret,
    )(xs, ys, zs)


def _knn_idx_tc(xyz, ct, K, gt=256, interpret=False):
    B, N, _ = xyz.shape
    G = ct.shape[2]

    def body(p_ref, c_ref, out_ref, d_ref):
        p = p_ref[0]
        c = c_ref[0]
        k2 = jnp.sum(p * p, axis=1, keepdims=True)
        q2 = jnp.sum(c * c, axis=0, keepdims=True)
        dot = jax.lax.dot_general(
            p, c, dimension_numbers=(((1,), (0,)), ((), ())),
            preferred_element_type=jnp.float32)
        d2 = q2 + k2 - 2.0 * dot
        d_ref[...] = jnp.sqrt(jnp.maximum(d2, 0.0))
        iota_n = lax.broadcasted_iota(jnp.int32, (N, gt), 0)
        iota_k = lax.broadcasted_iota(jnp.int32, (K, gt), 0)

        def step(k, acc):
            d_cur = d_ref[...]
            m = jnp.min(d_cur, axis=0, keepdims=True)
            hit = d_cur == m
            idx = jnp.min(jnp.where(hit, iota_n, N), axis=0,
                          keepdims=True)
            acc = acc + jnp.where(iota_k == k, idx, 0)
            d_ref[...] = jnp.where(iota_n == idx, jnp.inf, d_cur)
            return acc

        acc0 = jnp.zeros((K, gt), jnp.int32)
        out_ref[0] = lax.fori_loop(0, K, step, acc0)

    return pl.pallas_call(
        body,
        grid=(B, G // gt),
        in_specs=[
            pl.BlockSpec((1, N, 3), lambda b, g: (b, 0, 0)),
            pl.BlockSpec((1, 3, gt), lambda b, g: (b, 0, g)),
        ],
        out_specs=pl.BlockSpec((1, K, gt), lambda b, g: (b, 0, g)),
        out_shape=jax.ShapeDtypeStruct((B, K, G), jnp.int32),
        scratch_shapes=[pltpu.VMEM((N, gt), jnp.float32)],
        compiler_params=pltpu.CompilerParams(
            vmem_limit_bytes=110 * 1024 * 1024),
        interpret=interpret,
    )(xyz, ct)


def kernel(xyz, features):
    B, N, _ = xyz.shape
    C = features.shape[-1]
    G, K = _NUM_GROUPS, _GROUP_SIZE

    xt = xyz.transpose(0, 2, 1)
    cx, cy, cz = _fps_centers_tc(xt[:, 0], xt[:, 1], xt[:, 2], G)
    centers = jnp.stack([cx, cy, cz], axis=-1)
    ct = jnp.stack([cx, cy, cz], axis=1)
    knn_idx = _knn_idx_tc(xyz, ct, K).transpose(0, 2, 1)

    batch_offset = (jnp.arange(B, dtype=knn_idx.dtype) * N).reshape(-1, 1, 1)
    knn_idx_flat = (knn_idx + batch_offset).reshape(-1)

    xyzp = jnp.pad(xyz.reshape(-1, 3), ((0, 0), (0, 13)))
    nbr_feats, nbr_xyzp = _gather_rows_sc(
        features.reshape(-1, C), xyzp, knn_idx_flat)
    nbr_feats = nbr_feats.reshape(B, G, K, C)
    nbr_xyz = nbr_xyzp[:, :3].reshape(B, G, K, 3) - centers[:, :, None, :]
    group_feats = jnp.concatenate([nbr_xyz, nbr_feats], axis=-1)
    return group_feats, centers, knn_idx


# speedup vs baseline: 9.3282x; 1.0465x over previous
import functools

import jax
import jax.numpy as jnp
from jax import lax
from jax.experimental import pallas as pl
from jax.experimental.pallas import tpu as pltpu
from jax.experimental.pallas import tpu_sc as plsc

_NUM_GROUPS = 512
_GROUP_SIZE = 32

_NC, _NS = 2, 16
_NW = _NC * _NS


def _gather_rows_sc(feats, xyzp, idx, chunk=512):
    M = idx.shape[0]
    Df, Dx = feats.shape[1], xyzp.shape[1]
    b_per_w = M // _NW
    nchunks = b_per_w // chunk
    assert b_per_w % chunk == 0 and M % _NW == 0
    mesh = plsc.VectorSubcoreMesh(core_axis_name="c", subcore_axis_name="s")

    @functools.partial(
        pl.kernel,
        out_type=[
            jax.ShapeDtypeStruct((M, Df), jnp.float32),
            jax.ShapeDtypeStruct((M, Dx), jnp.float32),
        ],
        mesh=mesh,
        scratch_types=[
            pltpu.VMEM((b_per_w,), jnp.int32),
            pltpu.VMEM((2, chunk, Df), jnp.float32),
            pltpu.VMEM((2, chunk, Dx), jnp.float32),
            pltpu.SemaphoreType.DMA,
            pltpu.SemaphoreType.DMA,
        ],
        compiler_params=pltpu.CompilerParams(use_tc_tiling_on_sc=False),
    )
    def k(feats_hbm, xyzp_hbm, idx_hbm, of_hbm, ox_hbm,
          idx_v, frows, xrows, semf, semx):
        wid = lax.axis_index("s") * _NC + lax.axis_index("c")
        base = wid * b_per_w
        pltpu.sync_copy(idx_hbm.at[pl.ds(base, b_per_w)], idx_v)
        copies = [None] * nchunks
        for c in range(nchunks):
            ix = idx_v.at[pl.ds(c * chunk, chunk)]
            buf = c % 2
            copies[c] = (
                pltpu.async_copy(feats_hbm.at[ix], frows.at[buf], semf),
                pltpu.async_copy(xyzp_hbm.at[ix], xrows.at[buf], semx),
            )
            if c > 0:
                prev = c - 1
                for cp in copies[prev]:
                    cp.wait()
                sl = pl.ds(base + prev * chunk, chunk)
                pltpu.sync_copy(frows.at[prev % 2], of_hbm.at[sl])
                pltpu.sync_copy(xrows.at[prev % 2], ox_hbm.at[sl])
        last = nchunks - 1
        for cp in copies[last]:
            cp.wait()
        sl = pl.ds(base + last * chunk, chunk)
        pltpu.sync_copy(frows.at[last % 2], of_hbm.at[sl])
        pltpu.sync_copy(xrows.at[last % 2], ox_hbm.at[sl])

    return k(feats, xyzp, idx)


def _fps_centers_tc(xs, ys, zs, G, interpret=False):
    B, N = xs.shape

    def body(xs_ref, ys_ref, zs_ref, cx_ref, cy_ref, cz_ref):
        xs_, ys_, zs_ = xs_ref[...], ys_ref[...], zs_ref[...]
        iota_n = lax.broadcasted_iota(jnp.int32, (B, N), 1)
        iota_g = lax.broadcasted_iota(jnp.int32, (B, G), 1)

        def step(i, state):
            dists, far, cxa, cya, cza = state
            mask = iota_n == far
            cx = jnp.sum(jnp.where(mask, xs_, 0.0), axis=1, keepdims=True)
            cy = jnp.sum(jnp.where(mask, ys_, 0.0), axis=1, keepdims=True)
            cz = jnp.sum(jnp.where(mask, zs_, 0.0), axis=1, keepdims=True)
            gm = iota_g == i
            cxa = cxa + jnp.where(gm, cx, 0.0)
            cya = cya + jnp.where(gm, cy, 0.0)
            cza = cza + jnp.where(gm, cz, 0.0)
            dx = xs_ - cx
            dy = ys_ - cy
            dz = zs_ - cz
            d = dx * dx + dy * dy + dz * dz
            dists = jnp.minimum(dists, d)
            m = jnp.max(dists, axis=1, keepdims=True)
            far = jnp.min(jnp.where(dists == m, iota_n, N), axis=1,
                          keepdims=True)
            return dists, far, cxa, cya, cza

        dists0 = jnp.full((B, N), jnp.inf, jnp.float32)
        far0 = jnp.zeros((B, 1), jnp.int32)
        z = jnp.zeros((B, G), jnp.float32)
        _, _, cxa, cya, cza = lax.fori_loop(0, G, step,
                                            (dists0, far0, z, z, z))
        cx_ref[...] = cxa
        cy_ref[...] = cya
        cz_ref[...] = cza

    return pl.pallas_call(
        body,
        out_shape=[jax.ShapeDtypeStruct((B, G), jnp.float32)] * 3,
        interpret=interpret,
    )(xs, ys, zs)


def _knn_idx_tc(xyz, ct, K, gt=512, interpret=False):
    B, N, _ = xyz.shape
    G = ct.shape[2]

    def body(p_ref, c_ref, out_ref, d_ref):
        p = p_ref[0]
        c = c_ref[0]
        k2 = jnp.sum(p * p, axis=1, keepdims=True)
        q2 = jnp.sum(c * c, axis=0, keepdims=True)
        dot = jax.lax.dot_general(
            p, c, dimension_numbers=(((1,), (0,)), ((), ())),
            preferred_element_type=jnp.float32)
        d2 = q2 + k2 - 2.0 * dot
        d_ref[...] = jnp.sqrt(jnp.maximum(d2, 0.0))
        iota_n = lax.broadcasted_iota(jnp.int32, (N, gt), 0)
        iota_k = lax.broadcasted_iota(jnp.int32, (K, gt), 0)

        def step(k, acc):
            d_cur = d_ref[...]
            m = jnp.min(d_cur, axis=0, keepdims=True)
            hit = d_cur == m
            idx = jnp.min(jnp.where(hit, iota_n, N), axis=0,
                          keepdims=True)
            acc = acc + jnp.where(iota_k == k, idx, 0)
            d_ref[...] = jnp.where(iota_n == idx, jnp.inf, d_cur)
            return acc

        acc0 = jnp.zeros((K, gt), jnp.int32)
        out_ref[0] = lax.fori_loop(0, K, step, acc0)

    return pl.pallas_call(
        body,
        grid=(B, G // gt),
        in_specs=[
            pl.BlockSpec((1, N, 3), lambda b, g: (b, 0, 0)),
            pl.BlockSpec((1, 3, gt), lambda b, g: (b, 0, g)),
        ],
        out_specs=pl.BlockSpec((1, K, gt), lambda b, g: (b, 0, g)),
        out_shape=jax.ShapeDtypeStruct((B, K, G), jnp.int32),
        scratch_shapes=[pltpu.VMEM((N, gt), jnp.float32)],
        compiler_params=pltpu.CompilerParams(
            vmem_limit_bytes=110 * 1024 * 1024),
        interpret=interpret,
    )(xyz, ct)


def kernel(xyz, features):
    B, N, _ = xyz.shape
    C = features.shape[-1]
    G, K = _NUM_GROUPS, _GROUP_SIZE

    xt = xyz.transpose(0, 2, 1)
    cx, cy, cz = _fps_centers_tc(xt[:, 0], xt[:, 1], xt[:, 2], G)
    centers = jnp.stack([cx, cy, cz], axis=-1)
    ct = jnp.stack([cx, cy, cz], axis=1)
    knn_idx = _knn_idx_tc(xyz, ct, K).transpose(0, 2, 1)

    batch_offset = (jnp.arange(B, dtype=knn_idx.dtype) * N).reshape(-1, 1, 1)
    knn_idx_flat = (knn_idx + batch_offset).reshape(-1)

    xyzp = jnp.pad(xyz.reshape(-1, 3), ((0, 0), (0, 13)))
    nbr_feats, nbr_xyzp = _gather_rows_sc(
        features.reshape(-1, C), xyzp, knn_idx_flat)
    nbr_feats = nbr_feats.reshape(B, G, K, C)
    nbr_xyz = nbr_xyzp[:, :3].reshape(B, G, K, 3) - centers[:, :, None, :]
    group_feats = jnp.concatenate([nbr_xyz, nbr_feats], axis=-1)
    return group_feats, centers, knn_idx
